# baseline (speedup 1.0000x reference)
import functools

import jax
import jax.numpy as jnp
from jax import lax
from jax.experimental import pallas as pl
from jax.experimental.pallas import tpu as pltpu
from jax.experimental.pallas import tpu_sc as plsc

D_MODEL = 1024
SEQ_LEN = 8192
BATCH = 4

NUM_CORES = 2
NUM_SUBCORES = 16
NUM_WORKERS = NUM_CORES * NUM_SUBCORES
ROWS_PER_WORKER = SEQ_LEN // NUM_WORKERS
CHUNK_ROWS = 32
NUM_CHUNKS = ROWS_PER_WORKER // CHUNK_ROWS

_mesh = plsc.VectorSubcoreMesh(core_axis_name="c", subcore_axis_name="s")


@functools.partial(
    pl.kernel,
    mesh=_mesh,
    out_type=jax.ShapeDtypeStruct((BATCH, SEQ_LEN, D_MODEL), jnp.float32),
    scratch_types=[
        pltpu.VMEM_SHARED((NUM_SUBCORES, CHUNK_ROWS, D_MODEL), jnp.float32),
        pltpu.VMEM_SHARED((NUM_SUBCORES, CHUNK_ROWS, D_MODEL), jnp.float32),
        pltpu.SemaphoreType.DMA,
        pltpu.SemaphoreType.DMA,
        pltpu.SemaphoreType.DMA,
        pltpu.SemaphoreType.DMA,
    ],
)
def _broadcast_table(table_hbm, out_hbm, shared0, shared1, lsem0, lsem1, ssem0, ssem1):
    sid = lax.axis_index("s")
    wid = sid * NUM_CORES + lax.axis_index("c")
    base = wid * ROWS_PER_WORKER
    bufs = (shared0.at[sid], shared1.at[sid])
    lsems = (lsem0, lsem1)
    ssems = (ssem0, ssem1)

    def start_load(g, i):
        return pltpu.async_copy(
            table_hbm.at[pl.ds(base + g * CHUNK_ROWS, CHUNK_ROWS)], bufs[i], lsems[i]
        )

    loads = [start_load(0, 0), start_load(1, 1)]
    stores = [[], []]
    for g in range(NUM_CHUNKS):
        i = g % 2
        loads[i].wait()
        r0 = base + g * CHUNK_ROWS
        stores[i] = [
            pltpu.async_copy(bufs[i], out_hbm.at[b, pl.ds(r0, CHUNK_ROWS)], ssems[i])
            for b in range(BATCH)
        ]
        if g + 2 < NUM_CHUNKS:
            for st in stores[i]:
                st.wait()
            loads[i] = start_load(g + 2, i)
    for i in range(2):
        for st in stores[i]:
            st.wait()


def kernel(x, pos_table):
    del x
    return _broadcast_table(pos_table)


# speedup vs baseline: 1.1036x; 1.1036x over previous
import functools

import jax
import jax.numpy as jnp
from jax import lax
from jax.experimental import pallas as pl
from jax.experimental.pallas import tpu as pltpu
from jax.experimental.pallas import tpu_sc as plsc

D_MODEL = 1024
SEQ_LEN = 8192
BATCH = 4

NUM_CORES = 2
NUM_SUBCORES = 16
NUM_WORKERS = NUM_CORES * NUM_SUBCORES
ROWS_PER_WORKER = SEQ_LEN // NUM_WORKERS
CHUNK_ROWS = 32
NUM_CHUNKS = ROWS_PER_WORKER // CHUNK_ROWS

_mesh = plsc.VectorSubcoreMesh(core_axis_name="c", subcore_axis_name="s")


@functools.partial(
    pl.kernel,
    mesh=_mesh,
    out_type=jax.ShapeDtypeStruct((BATCH, SEQ_LEN, D_MODEL), jnp.float32),
    scratch_types=[
        pltpu.VMEM((CHUNK_ROWS, D_MODEL), jnp.float32),
        pltpu.VMEM((CHUNK_ROWS, D_MODEL), jnp.float32),
        pltpu.VMEM_SHARED((NUM_SUBCORES, CHUNK_ROWS, D_MODEL), jnp.float32),
        pltpu.VMEM_SHARED((NUM_SUBCORES, CHUNK_ROWS, D_MODEL), jnp.float32),
        pltpu.SemaphoreType.DMA,
        pltpu.SemaphoreType.DMA,
        pltpu.SemaphoreType.DMA,
        pltpu.SemaphoreType.DMA,
        pltpu.SemaphoreType.DMA,
        pltpu.SemaphoreType.DMA,
        pltpu.SemaphoreType.DMA,
        pltpu.SemaphoreType.DMA,
    ],
)
def _broadcast_table(table_hbm, out_hbm, tbuf0, tbuf1, sh0, sh1, *sems):
    sid = lax.axis_index("s")
    wid = sid * NUM_CORES + lax.axis_index("c")
    base = wid * ROWS_PER_WORKER
    bufs = (tbuf0, sh0.at[sid], tbuf1, sh1.at[sid])
    NBUF = len(bufs)
    lsems = sems[:NBUF]
    ssems = sems[NBUF:]

    def start_load(g, i):
        return pltpu.async_copy(
            table_hbm.at[pl.ds(base + g * CHUNK_ROWS, CHUNK_ROWS)], bufs[i], lsems[i]
        )

    loads = [start_load(g, g) for g in range(NBUF)]
    stores = [[] for _ in range(NBUF)]
    for g in range(NUM_CHUNKS):
        i = g % NBUF
        loads[i].wait()
        r0 = base + g * CHUNK_ROWS
        stores[i] = [
            pltpu.async_copy(bufs[i], out_hbm.at[b, pl.ds(r0, CHUNK_ROWS)], ssems[i])
            for b in range(BATCH)
        ]
        if g + NBUF < NUM_CHUNKS:
            for st in stores[i]:
                st.wait()
            loads[i] = start_load(g + NBUF, i)
    for i in range(NBUF):
        for st in stores[i]:
            st.wait()


def kernel(x, pos_table):
    del x
    return _broadcast_table(pos_table)


# speedup vs baseline: 1.1899x; 1.0782x over previous
import functools

import jax
import jax.numpy as jnp
from jax import lax
from jax.experimental import pallas as pl
from jax.experimental.pallas import tpu as pltpu
from jax.experimental.pallas import tpu_sc as plsc

D_MODEL = 1024
SEQ_LEN = 8192
BATCH = 4

NUM_CORES = 2
NUM_SUBCORES = 16
NUM_WORKERS = NUM_CORES * NUM_SUBCORES
ROWS_PER_WORKER = SEQ_LEN // NUM_WORKERS
CHUNK_ROWS = 32
NUM_CHUNKS = ROWS_PER_WORKER // CHUNK_ROWS

_mesh = plsc.VectorSubcoreMesh(core_axis_name="c", subcore_axis_name="s")


@functools.partial(
    pl.kernel,
    mesh=_mesh,
    out_type=jax.ShapeDtypeStruct((BATCH, SEQ_LEN, D_MODEL), jnp.float32),
    scratch_types=[
        pltpu.VMEM((CHUNK_ROWS, D_MODEL), jnp.float32),
        pltpu.VMEM((CHUNK_ROWS, D_MODEL), jnp.float32),
        pltpu.VMEM((CHUNK_ROWS, D_MODEL), jnp.float32),
        pltpu.SemaphoreType.DMA,
        pltpu.SemaphoreType.DMA,
        pltpu.SemaphoreType.DMA,
        pltpu.SemaphoreType.DMA,
        pltpu.SemaphoreType.DMA,
        pltpu.SemaphoreType.DMA,
    ],
)
def _broadcast_table(table_hbm, out_hbm, tbuf0, tbuf1, tbuf2, *sems):
    wid = lax.axis_index("s") * NUM_CORES + lax.axis_index("c")
    base = wid * ROWS_PER_WORKER
    bufs = (tbuf0, tbuf1, tbuf2)
    NBUF = len(bufs)
    lsems = sems[:NBUF]
    ssems = sems[NBUF:]

    def start_load(g, i):
        return pltpu.async_copy(
            table_hbm.at[pl.ds(base + g * CHUNK_ROWS, CHUNK_ROWS)], bufs[i], lsems[i]
        )

    loads = [start_load(g, g) for g in range(NBUF)]
    stores = [[] for _ in range(NBUF)]
    for g in range(NUM_CHUNKS):
        i = g % NBUF
        loads[i].wait()
        r0 = base + g * CHUNK_ROWS
        stores[i] = [
            pltpu.async_copy(bufs[i], out_hbm.at[b, pl.ds(r0, CHUNK_ROWS)], ssems[i])
            for b in range(BATCH)
        ]
        if g + NBUF < NUM_CHUNKS:
            for st in stores[i]:
                st.wait()
            loads[i] = start_load(g + NBUF, i)
    for i in range(NBUF):
        for st in stores[i]:
            st.wait()


def kernel(x, pos_table):
    del x
    return _broadcast_table(pos_table)
